# Initial kernel scaffold; baseline (speedup 1.0000x reference)
import jax
import jax.numpy as jnp
from jax.experimental import pallas as pl


def kernel(x, L, R):
    raise NotImplementedError("write your pallas kernel here")


# speedup vs baseline: 3.2508x; 3.2508x over previous
import functools

import jax
import jax.numpy as jnp
from jax import lax
from jax.experimental import pallas as pl
from jax.experimental.pallas import tpu as pltpu
from jax.experimental.pallas import tpu_sc as plsc


def _sc_gather(table, idx, n, k):
    info = plsc.get_sparse_core_info()
    nw = info.num_cores * info.num_subcores
    b_per_w = n // nw

    mesh = plsc.VectorSubcoreMesh(core_axis_name="c", subcore_axis_name="s")

    @functools.partial(
        pl.kernel,
        mesh=mesh,
        compiler_params=pltpu.CompilerParams(use_tc_tiling_on_sc=False),
        out_type=jax.ShapeDtypeStruct((n, k), jnp.float32),
        scratch_types=[
            pltpu.VMEM((b_per_w,), jnp.int32),
            pltpu.VMEM((b_per_w, k), jnp.float32),
            pltpu.SemaphoreType.DMA,
        ],
    )
    def gather_kernel(table_hbm, idx_hbm, out_hbm, idx_v, rows_v, sem):
        wid = lax.axis_index("s") * info.num_cores + lax.axis_index("c")
        base = wid * b_per_w
        pltpu.sync_copy(idx_hbm.at[pl.ds(base, b_per_w)], idx_v)
        pltpu.async_copy(table_hbm.at[idx_v], rows_v, sem).wait()
        pltpu.sync_copy(rows_v, out_hbm.at[pl.ds(base, b_per_w)])

    return gather_kernel(table, idx)


def _tc_matmul(g, r, n, k, d, block_m=8192):
    def mm_body(g_ref, r_ref, o_ref):
        o_ref[...] = lax.dot_general(
            g_ref[...], r_ref[...],
            (((1,), (1,)), ((), ())),
            preferred_element_type=jnp.float32,
        )

    return pl.pallas_call(
        mm_body,
        grid=(n // block_m,),
        in_specs=[
            pl.BlockSpec((block_m, k), lambda i: (i, 0)),
            pl.BlockSpec((d, k), lambda i: (0, 0)),
        ],
        out_specs=pl.BlockSpec((block_m, d), lambda i: (i, 0)),
        out_shape=jax.ShapeDtypeStruct((n, d), jnp.float32),
    )(g, r)


def kernel(x, L, R):
    b, h = x.shape
    v, k = L.shape
    d, _ = R.shape
    n = b * h

    idx = x.reshape(n).astype(jnp.int32)
    g = _sc_gather(L, idx, n, k)
    out = _tc_matmul(g, R, n, k, d)
    return out.reshape(b, h, d)


# speedup vs baseline: 5.0676x; 1.5589x over previous
import functools

import jax
import jax.numpy as jnp
from jax import lax
from jax.experimental import pallas as pl
from jax.experimental.pallas import tpu as pltpu
from jax.experimental.pallas import tpu_sc as plsc


def _sc_gather(table, idx, n, k):
    info = plsc.get_sparse_core_info()
    nw = info.num_cores * info.num_subcores
    b_per_w = n // nw

    mesh = plsc.VectorSubcoreMesh(core_axis_name="c", subcore_axis_name="s")

    @functools.partial(
        pl.kernel,
        mesh=mesh,
        compiler_params=pltpu.CompilerParams(use_tc_tiling_on_sc=False),
        out_type=jax.ShapeDtypeStruct((n, k), jnp.float32),
        scratch_types=[
            pltpu.VMEM((b_per_w,), jnp.int32),
            pltpu.VMEM((b_per_w, k), jnp.float32),
            pltpu.SemaphoreType.DMA,
        ],
    )
    def gather_kernel(table_hbm, idx_hbm, out_hbm, idx_v, rows_v, sem):
        wid = lax.axis_index("s") * info.num_cores + lax.axis_index("c")
        base = wid * b_per_w
        pltpu.sync_copy(idx_hbm.at[pl.ds(base, b_per_w)], idx_v)
        pltpu.async_copy(table_hbm.at[idx_v], rows_v, sem).wait()
        pltpu.sync_copy(rows_v, out_hbm.at[pl.ds(base, b_per_w)])

    return gather_kernel(table, idx)


def _tc_matmul(g128, r_big, b, h, k, d, bb=32):
    pack = 128 // k
    n_lines = b * h // pack
    block_m = 1600

    def mm_body(g_ref, r_ref, o_ref):
        o_ref[...] = jnp.dot(g_ref[...], r_ref[...],
                             preferred_element_type=jnp.float32)

    out512 = pl.pallas_call(
        mm_body,
        grid=(n_lines // block_m,),
        in_specs=[
            pl.BlockSpec((block_m, 128), lambda i: (i, 0)),
            pl.BlockSpec((128, pack * d), lambda i: (0, 0)),
        ],
        out_specs=pl.BlockSpec((block_m, pack * d), lambda i: (i, 0)),
        out_shape=jax.ShapeDtypeStruct((n_lines, pack * d), jnp.float32),
    )(g128, r_big)
    return out512.reshape(b, h, d)


def kernel(x, L, R):
    b, h = x.shape
    v, k = L.shape
    d, _ = R.shape
    n = b * h
    pack = 128 // k

    idx = x.reshape(n).astype(jnp.int32)
    g = _sc_gather(L, idx, n, k)
    g128 = g.reshape(n // pack, 128)
    r_big = jnp.kron(jnp.eye(pack, dtype=jnp.float32), R.T)
    return _tc_matmul(g128, r_big, b, h, k, d)
